# baseline (speedup 1.0000x reference)
import functools

import jax
import jax.numpy as jnp
import numpy as np
from jax import lax
from jax.experimental import pallas as pl
from jax.experimental.pallas import tpu as pltpu
from jax.experimental.pallas import tpu_sc as plsc

B = 8
C = 64
N = 2048
K = 20
O = 64

T = 256
HB = B // 2
NUM_WORKERS = 32
PTS_PER_WORKER = (HB * N) // NUM_WORKERS
CP = 32
NCHUNK = PTS_PER_WORKER // CP
IDX_PER_CHUNK = CP * K
GATHERS_PER_CHUNK = IDX_PER_CHUNK // 128
IDX_ROWS = (HB * N * K) // 128


def _knn_uv_body(x_ref, w1_ref, wd_ref, bias_ref,
                 idx_ref, u_ref, v_ref):
    b = pl.program_id(0)
    x_b = x_ref[0]
    x_t = x_ref[0, :, pl.ds(pl.program_id(1) * T, T)]

    sq = jnp.sum(x_b * x_b, axis=0, keepdims=True)
    inner = lax.dot_general(x_t, x_b, (((0,), (0,)), ((), ())),
                            preferred_element_type=jnp.float32)
    d = sq - 2.0 * inner

    u_ref[...] = lax.dot_general(
        x_t, w1_ref[...], (((0,), (0,)), ((), ())),
        preferred_element_type=jnp.float32).astype(jnp.bfloat16)
    v_ref[...] = (lax.dot_general(
        x_t, wd_ref[...], (((0,), (0,)), ((), ())),
        preferred_element_type=jnp.float32)
        + bias_ref[...][0:1]).astype(jnp.bfloat16)

    t = pl.program_id(1)
    lane_i = lax.broadcasted_iota(jnp.int32, (T, 128), 1)
    lane_f = lane_i.astype(jnp.float32)
    row2d = lax.broadcasted_iota(jnp.int32, (T, 128), 0)
    inf = jnp.float32(np.inf)
    s1 = jnp.full((T, 128), np.inf, jnp.float32)
    s2 = jnp.full((T, 128), np.inf, jnp.float32)
    s3 = jnp.full((T, 128), np.inf, jnp.float32)
    diagref = row2d - lane_i
    for c in range(N // 128):
        v = d[:, c * 128:(c + 1) * 128]
        vb = lax.bitcast_convert_type(v, jnp.int32)
        vp = lax.bitcast_convert_type((vb & ~jnp.int32(15)) | jnp.int32(c),
                                      jnp.float32)
        vp = jnp.where(diagref == (c - 2 * t) * 128, inf, vp)
        lo = jnp.minimum(s1, vp)
        vp = jnp.maximum(s1, vp)
        s1 = lo
        lo = jnp.minimum(s2, vp)
        vp = jnp.maximum(s2, vp)
        s2 = lo
        s3 = jnp.minimum(s3, vp)
    big = jnp.float32(3e8)
    nself = (b * N + t * T
             + lax.broadcasted_iota(jnp.int32, (T, 1), 0))
    cols = [nself]
    base_i = b * N
    for _ in range(K - 1):
        m = jnp.min(s1, axis=1, keepdims=True)
        eq = s1 == m
        lane = jnp.min(jnp.where(eq, lane_f, big), axis=1,
                       keepdims=True).astype(jnp.int32)
        chunk = lax.bitcast_convert_type(m, jnp.int32) & jnp.int32(15)
        cols.append(base_i + chunk * 128 + lane)
        s1 = jnp.where(eq, s2, s1)
        s2 = jnp.where(eq, s3, s2)
        s3 = jnp.where(eq, inf, s3)
    idx_ref[0] = jnp.concatenate(cols, axis=1)


def _knn_uv(x, w1t, wdt, bias8):
    return pl.pallas_call(
        _knn_uv_body,
        grid=(HB, N // T),
        in_specs=[
            pl.BlockSpec((1, C, N), lambda b, t: (b, 0, 0)),
            pl.BlockSpec((C, O), lambda b, t: (0, 0)),
            pl.BlockSpec((C, O), lambda b, t: (0, 0)),
            pl.BlockSpec((8, O), lambda b, t: (0, 0)),
        ],
        out_specs=[
            pl.BlockSpec((1, T, K), lambda b, t: (b, t, 0)),
            pl.BlockSpec((T, O), lambda b, t: (b * (N // T) + t, 0)),
            pl.BlockSpec((T, O), lambda b, t: (b * (N // T) + t, 0)),
        ],
        out_shape=[
            jax.ShapeDtypeStruct((HB, N, K), jnp.int32),
            jax.ShapeDtypeStruct((HB * N, O), jnp.bfloat16),
            jax.ShapeDtypeStruct((HB * N, O), jnp.bfloat16),
        ],
    )(x, w1t, wdt, bias8)


def _sc_gather_max_body(u_hbm, v_hbm, idx_hbm, out_hbm,
                        idx_v, rows_v, v_v, out_v, sems):
    wid = lax.axis_index("s") * 2 + lax.axis_index("c")
    rows_per_worker = (PTS_PER_WORKER * K) // 128
    pltpu.sync_copy(idx_hbm.at[pl.ds(wid * rows_per_worker, rows_per_worker)],
                    idx_v)

    def fire(c, buf):
        pbase = wid * PTS_PER_WORKER + c * CP
        copies = [
            pltpu.make_async_copy(
                u_hbm.at[idx_v.at[c * GATHERS_PER_CHUNK + g]],
                rows_v.at[buf].at[pl.ds(g * 128, 128)], sems.at[buf])
            for g in range(GATHERS_PER_CHUNK)
        ]
        copies.append(pltpu.make_async_copy(
            v_hbm.at[pl.ds(pbase, CP)], v_v.at[buf], sems.at[buf]))
        for cp in copies:
            cp.start()
        return copies

    def compute_store(c, buf):
        pbase = wid * PTS_PER_WORKER + c * CP

        def point_body(p, c2):
            for g2 in range(O // 32):
                cs = pl.ds(g2 * 32, 32)
                acc = rows_v[buf, p * K, cs]
                for j in range(1, K):
                    acc = jnp.maximum(acc, rows_v[buf, p * K + j, cs])
                t = acc + v_v[buf, p, cs]
                out_v[buf, p, cs] = jnp.maximum(t, jnp.bfloat16(0.2) * t)
            return c2

        lax.fori_loop(0, CP, point_body, 0)
        pltpu.sync_copy(out_v.at[buf], out_hbm.at[pl.ds(pbase, CP)])

    inflight = fire(0, 0)
    for c in range(NCHUNK):
        buf = c % 2
        if c + 1 < NCHUNK:
            nxt = fire(c + 1, 1 - buf)
        for cp in inflight:
            cp.wait()
        compute_store(c, buf)
        if c + 1 < NCHUNK:
            inflight = nxt


@functools.cache
def _sc_gather_max():
    return pl.kernel(
        _sc_gather_max_body,
        out_type=jax.ShapeDtypeStruct((HB * N, O), jnp.bfloat16),
        mesh=plsc.VectorSubcoreMesh(core_axis_name="c", subcore_axis_name="s"),
        compiler_params=pltpu.CompilerParams(use_tc_tiling_on_sc=False),
        scratch_types=[
            pltpu.VMEM(((PTS_PER_WORKER * K) // 128, 128), jnp.int32),
            pltpu.VMEM((2, IDX_PER_CHUNK, O), jnp.bfloat16),
            pltpu.VMEM((2, CP, O), jnp.bfloat16),
            pltpu.VMEM((2, CP, O), jnp.bfloat16),
            pltpu.SemaphoreType.DMA((2,)),
        ],
    )


@jax.jit
def kernel(x, W, gamma, beta, running_mean, running_var):
    scale = gamma / jnp.sqrt(running_var + 1e-5)
    bias = beta - running_mean * scale
    Wq = W * scale[:, None]
    w1t = jnp.transpose(Wq[:, :C])
    wdt = jnp.transpose(Wq[:, C:] - Wq[:, :C])
    bias8 = jnp.broadcast_to(bias[None, :], (8, O))

    outs = []
    for h in range(2):
        xh = lax.slice_in_dim(x, h * HB, (h + 1) * HB, axis=0)
        idx, u_flat, v_flat = _knn_uv(xh, w1t, wdt, bias8)
        idx_flat = idx.reshape(IDX_ROWS, 128)
        outs.append(_sc_gather_max()(u_flat, v_flat, idx_flat))
    out_t = jnp.concatenate(outs, axis=0)
    return jnp.transpose(out_t.reshape(B, N, O), (0, 2, 1)).astype(jnp.float32)


# speedup vs baseline: 1.0634x; 1.0634x over previous
import functools

import jax
import jax.numpy as jnp
import numpy as np
from jax import lax
from jax.experimental import pallas as pl
from jax.experimental.pallas import tpu as pltpu
from jax.experimental.pallas import tpu_sc as plsc

B = 8
C = 64
N = 2048
K = 20
O = 64

T = 256
NUM_WORKERS = 32
PTS_PER_WORKER = (B * N) // NUM_WORKERS
CP = 32
NCHUNK = PTS_PER_WORKER // CP
IDX_PER_CHUNK = CP * K
GATHERS_PER_CHUNK = IDX_PER_CHUNK // 128
IDX_ROWS = (B * N * K) // 128


def _knn_uv_body(x_ref, w1_ref, wd_ref, bias_ref,
                 idx_ref, u_ref, v_ref):
    b = pl.program_id(0)
    x_b = x_ref[0]
    x_t = x_ref[0, :, pl.ds(pl.program_id(1) * T, T)]

    sq = jnp.sum(x_b * x_b, axis=0, keepdims=True)
    inner = lax.dot_general(x_t * jnp.float32(-2.0), x_b,
                            (((0,), (0,)), ((), ())),
                            preferred_element_type=jnp.float32)
    d = sq + inner

    u_ref[...] = lax.dot_general(
        x_t, w1_ref[...], (((0,), (0,)), ((), ())),
        preferred_element_type=jnp.float32).astype(jnp.bfloat16)
    v_ref[...] = (lax.dot_general(
        x_t, wd_ref[...], (((0,), (0,)), ((), ())),
        preferred_element_type=jnp.float32)
        + bias_ref[...][0:1]).astype(jnp.bfloat16)

    t = pl.program_id(1)
    lane_i = lax.broadcasted_iota(jnp.int32, (T, 128), 1)
    lane_f = lane_i.astype(jnp.float32)
    row2d = lax.broadcasted_iota(jnp.int32, (T, 128), 0)
    inf = jnp.float32(np.inf)
    s1 = jnp.full((T, 128), np.inf, jnp.float32)
    s2 = jnp.full((T, 128), np.inf, jnp.float32)
    s3 = jnp.full((T, 128), np.inf, jnp.float32)
    diagref = row2d - lane_i
    for c in range(N // 128):
        v = d[:, c * 128:(c + 1) * 128]
        vb = lax.bitcast_convert_type(v, jnp.int32)
        vp = lax.bitcast_convert_type((vb & ~jnp.int32(15)) | jnp.int32(c),
                                      jnp.float32)
        vp = jnp.where(diagref == (c - 2 * t) * 128, inf, vp)
        lo = jnp.minimum(s1, vp)
        vp = jnp.maximum(s1, vp)
        s1 = lo
        lo = jnp.minimum(s2, vp)
        vp = jnp.maximum(s2, vp)
        s2 = lo
        s3 = jnp.minimum(s3, vp)
    big = jnp.float32(3e8)
    nself = (b * N + t * T
             + lax.broadcasted_iota(jnp.int32, (T, 1), 0))
    cols = [nself]
    base_i = b * N
    for _ in range(K - 1):
        m = jnp.min(s1, axis=1, keepdims=True)
        eq = s1 == m
        lane = jnp.min(jnp.where(eq, lane_f, big), axis=1,
                       keepdims=True).astype(jnp.int32)
        chunk = lax.bitcast_convert_type(m, jnp.int32) & jnp.int32(15)
        cols.append(base_i + chunk * 128 + lane)
        s1 = jnp.where(eq, s2, s1)
        s2 = jnp.where(eq, s3, s2)
        s3 = jnp.where(eq, inf, s3)
    idx_ref[0] = jnp.concatenate(cols, axis=1)


def _knn_uv(x, w1t, wdt, bias8):
    return pl.pallas_call(
        _knn_uv_body,
        grid=(B, N // T),
        in_specs=[
            pl.BlockSpec((1, C, N), lambda b, t: (b, 0, 0)),
            pl.BlockSpec((C, O), lambda b, t: (0, 0)),
            pl.BlockSpec((C, O), lambda b, t: (0, 0)),
            pl.BlockSpec((8, O), lambda b, t: (0, 0)),
        ],
        out_specs=[
            pl.BlockSpec((1, T, K), lambda b, t: (b, t, 0)),
            pl.BlockSpec((T, O), lambda b, t: (b * (N // T) + t, 0)),
            pl.BlockSpec((T, O), lambda b, t: (b * (N // T) + t, 0)),
        ],
        out_shape=[
            jax.ShapeDtypeStruct((B, N, K), jnp.int32),
            jax.ShapeDtypeStruct((B * N, O), jnp.bfloat16),
            jax.ShapeDtypeStruct((B * N, O), jnp.bfloat16),
        ],
    )(x, w1t, wdt, bias8)


def _sc_gather_max_body(u_hbm, v_hbm, idx_hbm, out_hbm,
                        idx_v, rows_v, v_v, out_v, sems):
    wid = lax.axis_index("s") * 2 + lax.axis_index("c")
    rows_per_worker = (PTS_PER_WORKER * K) // 128
    pltpu.sync_copy(idx_hbm.at[pl.ds(wid * rows_per_worker, rows_per_worker)],
                    idx_v)

    def fire(c, buf):
        pbase = wid * PTS_PER_WORKER + c * CP
        copies = [
            pltpu.make_async_copy(
                u_hbm.at[idx_v.at[c * GATHERS_PER_CHUNK + g]],
                rows_v.at[buf].at[pl.ds(g * 128, 128)], sems.at[buf])
            for g in range(GATHERS_PER_CHUNK)
        ]
        copies.append(pltpu.make_async_copy(
            v_hbm.at[pl.ds(pbase, CP)], v_v.at[buf], sems.at[buf]))
        for cp in copies:
            cp.start()
        return copies

    def compute_store(c, buf):
        pbase = wid * PTS_PER_WORKER + c * CP

        def point_body(p, c2):
            for g2 in range(O // 32):
                cs = pl.ds(g2 * 32, 32)
                acc = rows_v[buf, p * K, cs]
                for j in range(1, K):
                    acc = jnp.maximum(acc, rows_v[buf, p * K + j, cs])
                t = acc + v_v[buf, p, cs]
                out_v[buf, p, cs] = jnp.maximum(t, jnp.bfloat16(0.2) * t)
            return c2

        lax.fori_loop(0, CP, point_body, 0)
        pltpu.sync_copy(out_v.at[buf], out_hbm.at[pl.ds(pbase, CP)])

    inflight = fire(0, 0)
    for c in range(NCHUNK):
        buf = c % 2
        if c + 1 < NCHUNK:
            nxt = fire(c + 1, 1 - buf)
        for cp in inflight:
            cp.wait()
        compute_store(c, buf)
        if c + 1 < NCHUNK:
            inflight = nxt


@functools.cache
def _sc_gather_max():
    return pl.kernel(
        _sc_gather_max_body,
        out_type=jax.ShapeDtypeStruct((B * N, O), jnp.bfloat16),
        mesh=plsc.VectorSubcoreMesh(core_axis_name="c", subcore_axis_name="s"),
        compiler_params=pltpu.CompilerParams(use_tc_tiling_on_sc=False),
        scratch_types=[
            pltpu.VMEM(((PTS_PER_WORKER * K) // 128, 128), jnp.int32),
            pltpu.VMEM((2, IDX_PER_CHUNK, O), jnp.bfloat16),
            pltpu.VMEM((2, CP, O), jnp.bfloat16),
            pltpu.VMEM((2, CP, O), jnp.bfloat16),
            pltpu.SemaphoreType.DMA((2,)),
        ],
    )


@jax.jit
def kernel(x, W, gamma, beta, running_mean, running_var):
    scale = gamma / jnp.sqrt(running_var + 1e-5)
    bias = beta - running_mean * scale
    Wq = W * scale[:, None]
    w1t = jnp.transpose(Wq[:, :C])
    wdt = jnp.transpose(Wq[:, C:] - Wq[:, :C])
    bias8 = jnp.broadcast_to(bias[None, :], (8, O))

    idx, u_flat, v_flat = _knn_uv(x, w1t, wdt, bias8)
    idx_flat = idx.reshape(IDX_ROWS, 128)
    out_t = _sc_gather_max()(u_flat, v_flat, idx_flat)
    return jnp.transpose(out_t.reshape(B, N, O), (0, 2, 1)).astype(jnp.float32)


# speedup vs baseline: 1.1025x; 1.0368x over previous
import functools

import jax
import jax.numpy as jnp
import numpy as np
from jax import lax
from jax.experimental import pallas as pl
from jax.experimental.pallas import tpu as pltpu
from jax.experimental.pallas import tpu_sc as plsc

B = 8
C = 64
N = 2048
K = 20
O = 64

T = 512
NUM_WORKERS = 32
PTS_PER_WORKER = (B * N) // NUM_WORKERS
CP = 32
NCHUNK = PTS_PER_WORKER // CP
IDX_PER_CHUNK = CP * K
GATHERS_PER_CHUNK = IDX_PER_CHUNK // 128
IDX_ROWS = (B * N * K) // 128


def _knn_uv_body(x_ref, w1_ref, wd_ref, bias_ref,
                 idx_ref, u_ref, v_ref):
    b = pl.program_id(0)
    x_b = x_ref[0]
    x_t = x_ref[0, :, pl.ds(pl.program_id(1) * T, T)]

    sq = jnp.sum(x_b * x_b, axis=0, keepdims=True)
    inner = lax.dot_general(x_t * jnp.float32(-2.0), x_b,
                            (((0,), (0,)), ((), ())),
                            preferred_element_type=jnp.float32)
    d = sq + inner

    u_ref[...] = lax.dot_general(
        x_t, w1_ref[...], (((0,), (0,)), ((), ())),
        preferred_element_type=jnp.float32).astype(jnp.bfloat16)
    v_ref[...] = (lax.dot_general(
        x_t, wd_ref[...], (((0,), (0,)), ((), ())),
        preferred_element_type=jnp.float32)
        + bias_ref[...][0:1]).astype(jnp.bfloat16)

    t = pl.program_id(1)
    lane_i = lax.broadcasted_iota(jnp.int32, (T, 128), 1)
    lane_f = lane_i.astype(jnp.float32)
    row2d = lax.broadcasted_iota(jnp.int32, (T, 128), 0)
    inf = jnp.float32(np.inf)
    s1 = jnp.full((T, 128), np.inf, jnp.float32)
    s2 = jnp.full((T, 128), np.inf, jnp.float32)
    s3 = jnp.full((T, 128), np.inf, jnp.float32)
    diagref = row2d - lane_i
    for c in range(N // 128):
        v = d[:, c * 128:(c + 1) * 128]
        vb = lax.bitcast_convert_type(v, jnp.int32)
        vp = lax.bitcast_convert_type((vb & ~jnp.int32(15)) | jnp.int32(c),
                                      jnp.float32)
        vp = jnp.where(diagref == (c - 2 * t) * 128, inf, vp)
        lo = jnp.minimum(s1, vp)
        vp = jnp.maximum(s1, vp)
        s1 = lo
        lo = jnp.minimum(s2, vp)
        vp = jnp.maximum(s2, vp)
        s2 = lo
        s3 = jnp.minimum(s3, vp)
    big = jnp.float32(3e8)
    nself = (b * N + t * T
             + lax.broadcasted_iota(jnp.int32, (T, 1), 0))
    cols = [nself]
    base_i = b * N
    for _ in range(K - 1):
        m = jnp.min(s1, axis=1, keepdims=True)
        eq = s1 == m
        lane = jnp.min(jnp.where(eq, lane_f, big), axis=1,
                       keepdims=True).astype(jnp.int32)
        chunk = lax.bitcast_convert_type(m, jnp.int32) & jnp.int32(15)
        cols.append(base_i + chunk * 128 + lane)
        s1 = jnp.where(eq, s2, s1)
        s2 = jnp.where(eq, s3, s2)
        s3 = jnp.where(eq, inf, s3)
    idx_ref[0] = jnp.concatenate(cols, axis=1)


def _knn_uv(x, w1t, wdt, bias8):
    return pl.pallas_call(
        _knn_uv_body,
        grid=(B, N // T),
        in_specs=[
            pl.BlockSpec((1, C, N), lambda b, t: (b, 0, 0)),
            pl.BlockSpec((C, O), lambda b, t: (0, 0)),
            pl.BlockSpec((C, O), lambda b, t: (0, 0)),
            pl.BlockSpec((8, O), lambda b, t: (0, 0)),
        ],
        out_specs=[
            pl.BlockSpec((1, T, K), lambda b, t: (b, t, 0)),
            pl.BlockSpec((T, O), lambda b, t: (b * (N // T) + t, 0)),
            pl.BlockSpec((T, O), lambda b, t: (b * (N // T) + t, 0)),
        ],
        out_shape=[
            jax.ShapeDtypeStruct((B, N, K), jnp.int32),
            jax.ShapeDtypeStruct((B * N, O), jnp.bfloat16),
            jax.ShapeDtypeStruct((B * N, O), jnp.bfloat16),
        ],
    )(x, w1t, wdt, bias8)


def _sc_gather_max_body(u_hbm, v_hbm, idx_hbm, out_hbm,
                        idx_v, rows_v, v_v, out_v, sems):
    wid = lax.axis_index("s") * 2 + lax.axis_index("c")
    rows_per_worker = (PTS_PER_WORKER * K) // 128
    pltpu.sync_copy(idx_hbm.at[pl.ds(wid * rows_per_worker, rows_per_worker)],
                    idx_v)

    def fire(c, buf):
        pbase = wid * PTS_PER_WORKER + c * CP
        copies = [
            pltpu.make_async_copy(
                u_hbm.at[idx_v.at[c * GATHERS_PER_CHUNK + g]],
                rows_v.at[buf].at[pl.ds(g * 128, 128)], sems.at[buf])
            for g in range(GATHERS_PER_CHUNK)
        ]
        copies.append(pltpu.make_async_copy(
            v_hbm.at[pl.ds(pbase, CP)], v_v.at[buf], sems.at[buf]))
        for cp in copies:
            cp.start()
        return copies

    def compute_store(c, buf):
        pbase = wid * PTS_PER_WORKER + c * CP

        def point_body(p, c2):
            for g2 in range(O // 32):
                cs = pl.ds(g2 * 32, 32)
                acc = rows_v[buf, p * K, cs]
                for j in range(1, K):
                    acc = jnp.maximum(acc, rows_v[buf, p * K + j, cs])
                t = acc + v_v[buf, p, cs]
                out_v[buf, p, cs] = jnp.maximum(t, jnp.bfloat16(0.2) * t)
            return c2

        lax.fori_loop(0, CP, point_body, 0)
        pltpu.sync_copy(out_v.at[buf], out_hbm.at[pl.ds(pbase, CP)])

    inflight = fire(0, 0)
    for c in range(NCHUNK):
        buf = c % 2
        if c + 1 < NCHUNK:
            nxt = fire(c + 1, 1 - buf)
        for cp in inflight:
            cp.wait()
        compute_store(c, buf)
        if c + 1 < NCHUNK:
            inflight = nxt


@functools.cache
def _sc_gather_max():
    return pl.kernel(
        _sc_gather_max_body,
        out_type=jax.ShapeDtypeStruct((B * N, O), jnp.bfloat16),
        mesh=plsc.VectorSubcoreMesh(core_axis_name="c", subcore_axis_name="s"),
        compiler_params=pltpu.CompilerParams(use_tc_tiling_on_sc=False),
        scratch_types=[
            pltpu.VMEM(((PTS_PER_WORKER * K) // 128, 128), jnp.int32),
            pltpu.VMEM((2, IDX_PER_CHUNK, O), jnp.bfloat16),
            pltpu.VMEM((2, CP, O), jnp.bfloat16),
            pltpu.VMEM((2, CP, O), jnp.bfloat16),
            pltpu.SemaphoreType.DMA((2,)),
        ],
    )


@jax.jit
def kernel(x, W, gamma, beta, running_mean, running_var):
    scale = gamma / jnp.sqrt(running_var + 1e-5)
    bias = beta - running_mean * scale
    Wq = W * scale[:, None]
    w1t = jnp.transpose(Wq[:, :C])
    wdt = jnp.transpose(Wq[:, C:] - Wq[:, :C])
    bias8 = jnp.broadcast_to(bias[None, :], (8, O))

    idx, u_flat, v_flat = _knn_uv(x, w1t, wdt, bias8)
    idx_flat = idx.reshape(IDX_ROWS, 128)
    out_t = _sc_gather_max()(u_flat, v_flat, idx_flat)
    return jnp.transpose(out_t.reshape(B, N, O), (0, 2, 1)).astype(jnp.float32)


# speedup vs baseline: 1.1030x; 1.0004x over previous
import functools

import jax
import jax.numpy as jnp
import numpy as np
from jax import lax
from jax.experimental import pallas as pl
from jax.experimental.pallas import tpu as pltpu
from jax.experimental.pallas import tpu_sc as plsc

B = 8
C = 64
N = 2048
K = 20
O = 64

T = 512
NUM_WORKERS = 32
PTS_PER_WORKER = (B * N) // NUM_WORKERS
CP = 32
NCHUNK = PTS_PER_WORKER // CP
IDX_PER_CHUNK = CP * K
GATHERS_PER_CHUNK = IDX_PER_CHUNK // 128
IDX_ROWS = (B * N * K) // 128


def _knn_uv_body(x_ref, w1_ref, wd_ref, bias_ref,
                 idx_ref, u_ref, v_ref):
    b = pl.program_id(0)
    x_b = x_ref[0]
    x_t = x_ref[0, :, pl.ds(pl.program_id(1) * T, T)]

    sq = jnp.sum(x_b * x_b, axis=0, keepdims=True)
    inner = lax.dot_general(x_t * jnp.float32(-2.0), x_b,
                            (((0,), (0,)), ((), ())),
                            preferred_element_type=jnp.float32)
    d = sq + inner

    u_ref[...] = lax.dot_general(
        x_t, w1_ref[...], (((0,), (0,)), ((), ())),
        preferred_element_type=jnp.float32).astype(jnp.bfloat16)
    v_ref[...] = (lax.dot_general(
        x_t, wd_ref[...], (((0,), (0,)), ((), ())),
        preferred_element_type=jnp.float32)
        + bias_ref[...][0:1]).astype(jnp.bfloat16)

    t = pl.program_id(1)
    lane_i = lax.broadcasted_iota(jnp.int32, (T, 128), 1)
    lane_f = lane_i.astype(jnp.float32)
    row2d = lax.broadcasted_iota(jnp.int32, (T, 128), 0)
    inf = jnp.float32(np.inf)
    s1 = jnp.full((T, 128), np.inf, jnp.float32)
    s2 = jnp.full((T, 128), np.inf, jnp.float32)
    s3 = jnp.full((T, 128), np.inf, jnp.float32)
    diagref = row2d - lane_i
    for c in range(N // 128):
        v = d[:, c * 128:(c + 1) * 128]
        vb = lax.bitcast_convert_type(v, jnp.int32)
        vp = lax.bitcast_convert_type((vb & ~jnp.int32(15)) | jnp.int32(c),
                                      jnp.float32)
        vp = jnp.where(diagref == (c - (T // 128) * t) * 128, inf, vp)
        lo = jnp.minimum(s1, vp)
        vp = jnp.maximum(s1, vp)
        s1 = lo
        lo = jnp.minimum(s2, vp)
        vp = jnp.maximum(s2, vp)
        s2 = lo
        s3 = jnp.minimum(s3, vp)
    big = jnp.float32(3e8)
    nself = (b * N + t * T
             + lax.broadcasted_iota(jnp.int32, (T, 1), 0))
    cols = [nself]
    base_i = b * N
    for _ in range(K - 1):
        m = jnp.min(s1, axis=1, keepdims=True)
        eq = s1 == m
        lane = jnp.min(jnp.where(eq, lane_f, big), axis=1,
                       keepdims=True).astype(jnp.int32)
        chunk = lax.bitcast_convert_type(m, jnp.int32) & jnp.int32(15)
        cols.append(base_i + chunk * 128 + lane)
        s1 = jnp.where(eq, s2, s1)
        s2 = jnp.where(eq, s3, s2)
        s3 = jnp.where(eq, inf, s3)
    idx_ref[0] = jnp.concatenate(cols, axis=1)


def _knn_uv(x, w1t, wdt, bias8):
    return pl.pallas_call(
        _knn_uv_body,
        grid=(B, N // T),
        in_specs=[
            pl.BlockSpec((1, C, N), lambda b, t: (b, 0, 0)),
            pl.BlockSpec((C, O), lambda b, t: (0, 0)),
            pl.BlockSpec((C, O), lambda b, t: (0, 0)),
            pl.BlockSpec((8, O), lambda b, t: (0, 0)),
        ],
        out_specs=[
            pl.BlockSpec((1, T, K), lambda b, t: (b, t, 0)),
            pl.BlockSpec((T, O), lambda b, t: (b * (N // T) + t, 0)),
            pl.BlockSpec((T, O), lambda b, t: (b * (N // T) + t, 0)),
        ],
        out_shape=[
            jax.ShapeDtypeStruct((B, N, K), jnp.int32),
            jax.ShapeDtypeStruct((B * N, O), jnp.bfloat16),
            jax.ShapeDtypeStruct((B * N, O), jnp.bfloat16),
        ],
    )(x, w1t, wdt, bias8)


def _sc_gather_max_body(u_hbm, v_hbm, idx_hbm, out_hbm,
                        idx_v, rows_v, v_v, out_v, sems):
    wid = lax.axis_index("s") * 2 + lax.axis_index("c")
    rows_per_worker = (PTS_PER_WORKER * K) // 128
    pltpu.sync_copy(idx_hbm.at[pl.ds(wid * rows_per_worker, rows_per_worker)],
                    idx_v)

    def fire(c, buf):
        pbase = wid * PTS_PER_WORKER + c * CP
        copies = [
            pltpu.make_async_copy(
                u_hbm.at[idx_v.at[c * GATHERS_PER_CHUNK + g]],
                rows_v.at[buf].at[pl.ds(g * 128, 128)], sems.at[buf])
            for g in range(GATHERS_PER_CHUNK)
        ]
        copies.append(pltpu.make_async_copy(
            v_hbm.at[pl.ds(pbase, CP)], v_v.at[buf], sems.at[buf]))
        for cp in copies:
            cp.start()
        return copies

    def compute_store(c, buf):
        pbase = wid * PTS_PER_WORKER + c * CP

        def point_body(p, c2):
            for g2 in range(O // 32):
                cs = pl.ds(g2 * 32, 32)
                acc = rows_v[buf, p * K, cs]
                for j in range(1, K):
                    acc = jnp.maximum(acc, rows_v[buf, p * K + j, cs])
                t = acc + v_v[buf, p, cs]
                out_v[buf, p, cs] = jnp.maximum(t, jnp.bfloat16(0.2) * t)
            return c2

        lax.fori_loop(0, CP, point_body, 0)
        pltpu.sync_copy(out_v.at[buf], out_hbm.at[pl.ds(pbase, CP)])

    inflight = fire(0, 0)
    for c in range(NCHUNK):
        buf = c % 2
        if c + 1 < NCHUNK:
            nxt = fire(c + 1, 1 - buf)
        for cp in inflight:
            cp.wait()
        compute_store(c, buf)
        if c + 1 < NCHUNK:
            inflight = nxt


@functools.cache
def _sc_gather_max():
    return pl.kernel(
        _sc_gather_max_body,
        out_type=jax.ShapeDtypeStruct((B * N, O), jnp.bfloat16),
        mesh=plsc.VectorSubcoreMesh(core_axis_name="c", subcore_axis_name="s"),
        compiler_params=pltpu.CompilerParams(use_tc_tiling_on_sc=False),
        scratch_types=[
            pltpu.VMEM(((PTS_PER_WORKER * K) // 128, 128), jnp.int32),
            pltpu.VMEM((2, IDX_PER_CHUNK, O), jnp.bfloat16),
            pltpu.VMEM((2, CP, O), jnp.bfloat16),
            pltpu.VMEM((2, CP, O), jnp.bfloat16),
            pltpu.SemaphoreType.DMA((2,)),
        ],
    )


@jax.jit
def kernel(x, W, gamma, beta, running_mean, running_var):
    scale = gamma / jnp.sqrt(running_var + 1e-5)
    bias = beta - running_mean * scale
    Wq = W * scale[:, None]
    w1t = jnp.transpose(Wq[:, :C])
    wdt = jnp.transpose(Wq[:, C:] - Wq[:, :C])
    bias8 = jnp.broadcast_to(bias[None, :], (8, O))

    idx, u_flat, v_flat = _knn_uv(x, w1t, wdt, bias8)
    idx_flat = idx.reshape(IDX_ROWS, 128)
    out_t = _sc_gather_max()(u_flat, v_flat, idx_flat)
    return jnp.transpose(out_t.reshape(B, N, O), (0, 2, 1)).astype(jnp.float32)


# speedup vs baseline: 1.1281x; 1.0227x over previous
import functools

import jax
import jax.numpy as jnp
import numpy as np
from jax import lax
from jax.experimental import pallas as pl
from jax.experimental.pallas import tpu as pltpu
from jax.experimental.pallas import tpu_sc as plsc

B = 8
C = 64
N = 2048
K = 20
O = 64

T = 1024
NUM_WORKERS = 32
PTS_PER_WORKER = (B * N) // NUM_WORKERS
CP = 32
NCHUNK = PTS_PER_WORKER // CP
IDX_PER_CHUNK = CP * K
GATHERS_PER_CHUNK = IDX_PER_CHUNK // 128
IDX_ROWS = (B * N * K) // 128


def _knn_uv_body(x_ref, w1_ref, wd_ref, bias_ref,
                 idx_ref, u_ref, v_ref):
    b = pl.program_id(0)
    x_b = x_ref[0]
    x_t = x_ref[0, :, pl.ds(pl.program_id(1) * T, T)]

    sq = jnp.sum(x_b * x_b, axis=0, keepdims=True)
    inner = lax.dot_general(x_t * jnp.float32(-2.0), x_b,
                            (((0,), (0,)), ((), ())),
                            preferred_element_type=jnp.float32)
    d = sq + inner

    u_ref[...] = lax.dot_general(
        x_t, w1_ref[...], (((0,), (0,)), ((), ())),
        preferred_element_type=jnp.float32).astype(jnp.bfloat16)
    v_ref[...] = (lax.dot_general(
        x_t, wd_ref[...], (((0,), (0,)), ((), ())),
        preferred_element_type=jnp.float32)
        + bias_ref[...][0:1]).astype(jnp.bfloat16)

    t = pl.program_id(1)
    lane_i = lax.broadcasted_iota(jnp.int32, (T, 128), 1)
    lane_f = lane_i.astype(jnp.float32)
    row2d = lax.broadcasted_iota(jnp.int32, (T, 128), 0)
    inf = jnp.float32(np.inf)
    s1 = jnp.full((T, 128), np.inf, jnp.float32)
    s2 = jnp.full((T, 128), np.inf, jnp.float32)
    s3 = jnp.full((T, 128), np.inf, jnp.float32)
    diagref = row2d - lane_i
    for c in range(N // 128):
        v = d[:, c * 128:(c + 1) * 128]
        vb = lax.bitcast_convert_type(v, jnp.int32)
        vp = lax.bitcast_convert_type((vb & ~jnp.int32(15)) | jnp.int32(c),
                                      jnp.float32)
        vp = jnp.where(diagref == (c - (T // 128) * t) * 128, inf, vp)
        lo = jnp.minimum(s1, vp)
        vp = jnp.maximum(s1, vp)
        s1 = lo
        lo = jnp.minimum(s2, vp)
        vp = jnp.maximum(s2, vp)
        s2 = lo
        s3 = jnp.minimum(s3, vp)
    big = jnp.float32(3e8)
    nself = (b * N + t * T
             + lax.broadcasted_iota(jnp.int32, (T, 1), 0))
    cols = [nself]
    base_i = b * N
    for _ in range(K - 1):
        m = jnp.min(s1, axis=1, keepdims=True)
        eq = s1 == m
        lane = jnp.min(jnp.where(eq, lane_f, big), axis=1,
                       keepdims=True).astype(jnp.int32)
        chunk = lax.bitcast_convert_type(m, jnp.int32) & jnp.int32(15)
        cols.append(base_i + chunk * 128 + lane)
        s1 = jnp.where(eq, s2, s1)
        s2 = jnp.where(eq, s3, s2)
        s3 = jnp.where(eq, inf, s3)
    idx_ref[0] = jnp.concatenate(cols, axis=1)


def _knn_uv(x, w1t, wdt, bias8):
    return pl.pallas_call(
        _knn_uv_body,
        grid=(B, N // T),
        in_specs=[
            pl.BlockSpec((1, C, N), lambda b, t: (b, 0, 0)),
            pl.BlockSpec((C, O), lambda b, t: (0, 0)),
            pl.BlockSpec((C, O), lambda b, t: (0, 0)),
            pl.BlockSpec((8, O), lambda b, t: (0, 0)),
        ],
        out_specs=[
            pl.BlockSpec((1, T, K), lambda b, t: (b, t, 0)),
            pl.BlockSpec((T, O), lambda b, t: (b * (N // T) + t, 0)),
            pl.BlockSpec((T, O), lambda b, t: (b * (N // T) + t, 0)),
        ],
        out_shape=[
            jax.ShapeDtypeStruct((B, N, K), jnp.int32),
            jax.ShapeDtypeStruct((B * N, O), jnp.bfloat16),
            jax.ShapeDtypeStruct((B * N, O), jnp.bfloat16),
        ],
    )(x, w1t, wdt, bias8)


def _sc_gather_max_body(u_hbm, v_hbm, idx_hbm, out_hbm,
                        idx_v, rows_v, v_v, out_v, sems):
    wid = lax.axis_index("s") * 2 + lax.axis_index("c")
    rows_per_worker = (PTS_PER_WORKER * K) // 128
    pltpu.sync_copy(idx_hbm.at[pl.ds(wid * rows_per_worker, rows_per_worker)],
                    idx_v)

    def fire(c, buf):
        pbase = wid * PTS_PER_WORKER + c * CP
        copies = [
            pltpu.make_async_copy(
                u_hbm.at[idx_v.at[c * GATHERS_PER_CHUNK + g]],
                rows_v.at[buf].at[pl.ds(g * 128, 128)], sems.at[buf])
            for g in range(GATHERS_PER_CHUNK)
        ]
        copies.append(pltpu.make_async_copy(
            v_hbm.at[pl.ds(pbase, CP)], v_v.at[buf], sems.at[buf]))
        for cp in copies:
            cp.start()
        return copies

    def compute_store(c, buf):
        pbase = wid * PTS_PER_WORKER + c * CP

        def point_body(p, c2):
            for g2 in range(O // 32):
                cs = pl.ds(g2 * 32, 32)
                acc = rows_v[buf, p * K, cs]
                for j in range(1, K):
                    acc = jnp.maximum(acc, rows_v[buf, p * K + j, cs])
                t = acc + v_v[buf, p, cs]
                out_v[buf, p, cs] = jnp.maximum(t, jnp.bfloat16(0.2) * t)
            return c2

        lax.fori_loop(0, CP, point_body, 0)
        pltpu.sync_copy(out_v.at[buf], out_hbm.at[pl.ds(pbase, CP)])

    inflight = fire(0, 0)
    for c in range(NCHUNK):
        buf = c % 2
        if c + 1 < NCHUNK:
            nxt = fire(c + 1, 1 - buf)
        for cp in inflight:
            cp.wait()
        compute_store(c, buf)
        if c + 1 < NCHUNK:
            inflight = nxt


@functools.cache
def _sc_gather_max():
    return pl.kernel(
        _sc_gather_max_body,
        out_type=jax.ShapeDtypeStruct((B * N, O), jnp.bfloat16),
        mesh=plsc.VectorSubcoreMesh(core_axis_name="c", subcore_axis_name="s"),
        compiler_params=pltpu.CompilerParams(use_tc_tiling_on_sc=False),
        scratch_types=[
            pltpu.VMEM(((PTS_PER_WORKER * K) // 128, 128), jnp.int32),
            pltpu.VMEM((2, IDX_PER_CHUNK, O), jnp.bfloat16),
            pltpu.VMEM((2, CP, O), jnp.bfloat16),
            pltpu.VMEM((2, CP, O), jnp.bfloat16),
            pltpu.SemaphoreType.DMA((2,)),
        ],
    )


@jax.jit
def kernel(x, W, gamma, beta, running_mean, running_var):
    scale = gamma / jnp.sqrt(running_var + 1e-5)
    bias = beta - running_mean * scale
    Wq = W * scale[:, None]
    w1t = jnp.transpose(Wq[:, :C])
    wdt = jnp.transpose(Wq[:, C:] - Wq[:, :C])
    bias8 = jnp.broadcast_to(bias[None, :], (8, O))

    idx, u_flat, v_flat = _knn_uv(x, w1t, wdt, bias8)
    idx_flat = idx.reshape(IDX_ROWS, 128)
    out_t = _sc_gather_max()(u_flat, v_flat, idx_flat)
    return jnp.transpose(out_t.reshape(B, N, O), (0, 2, 1)).astype(jnp.float32)
